# Initial kernel scaffold; baseline (speedup 1.0000x reference)
import jax
import jax.numpy as jnp
from jax.experimental import pallas as pl


def kernel(x, W1, b1, W2, b2):
    raise NotImplementedError("write your pallas kernel here")


# speedup vs baseline: 4.7720x; 4.7720x over previous
import functools

import jax
import jax.numpy as jnp
from jax import lax
from jax.experimental import pallas as pl
from jax.experimental.pallas import tpu as pltpu
from jax.experimental.pallas import tpu_sc as plsc

N = 10000
D = 128
K = 20
RB = 200
NEG = -3e38

N_PAD = 10240
TILES = 32
SLOTS = 32
SRWS = 160
SGS = N_PAD // SRWS
DUMMY = N
FPT = D // TILES


def _topk_body(xn_blk_ref, xn_all_ref, vals_ref, idx_ref, a_ref):
    xn_blk = xn_blk_ref[...]
    xn_all = xn_all_ref[...]
    a_ref[...] = jax.lax.dot_general(
        xn_blk, xn_all, (((1,), (1,)), ((), ())),
        preferred_element_type=jnp.float32)
    iota = jax.lax.broadcasted_iota(jnp.int32, (RB, N), 1)
    kiota = jax.lax.broadcasted_iota(jnp.int32, (RB, K), 1)
    vals_ref[...] = jnp.zeros((RB, K), jnp.float32)
    idx_ref[...] = jnp.zeros((RB, K), jnp.int32)

    def _round(k, _):
        a = a_ref[...]
        m = jnp.max(a, axis=1)
        am = jnp.min(jnp.where(a >= m[:, None], iota, N), axis=1)
        vals_ref[...] = jnp.where(kiota == k, m[:, None], vals_ref[...])
        idx_ref[...] = jnp.where(kiota == k, am[:, None], idx_ref[...])
        a_ref[...] = jnp.where(iota == am[:, None], NEG, a)
        return 0

    jax.lax.fori_loop(0, K, _round, 0)


def _norm_body(x_ref, xn_ref):
    x = x_ref[...]
    nrm = jnp.sqrt(jnp.sum(x * x, axis=1, keepdims=True))
    xn_ref[...] = x / jnp.maximum(nrm, 1e-12)


def _knn_topk(x):
    xn = pl.pallas_call(
        _norm_body,
        out_shape=jax.ShapeDtypeStruct((N, D), jnp.float32),
        grid=(5,),
        in_specs=[pl.BlockSpec((N // 5, D), lambda i: (i, 0))],
        out_specs=pl.BlockSpec((N // 5, D), lambda i: (i, 0)),
    )(x)
    vals, idx = pl.pallas_call(
        _topk_body,
        out_shape=(jax.ShapeDtypeStruct((N, K), jnp.float32),
                   jax.ShapeDtypeStruct((N, K), jnp.int32)),
        grid=(N // RB,),
        in_specs=[pl.BlockSpec((RB, D), lambda i: (i, 0)),
                  pl.BlockSpec((N, D), lambda i: (0, 0))],
        out_specs=(pl.BlockSpec((RB, K), lambda i: (i, 0)),
                   pl.BlockSpec((RB, K), lambda i: (i, 0))),
        scratch_shapes=[pltpu.VMEM((RB, N), jnp.float32)],
    )(xn, xn)
    return vals, idx


def _mmT_body(w_ref, xT_ref, o_ref):
    o_ref[...] = jax.lax.dot_general(
        w_ref[...], xT_ref[...], (((0,), (0,)), ((), ())),
        preferred_element_type=jnp.float32)


def _mmT(xT, W):
    return pl.pallas_call(
        _mmT_body,
        out_shape=jax.ShapeDtypeStruct((D, N_PAD), jnp.float32),
        grid=(8,),
        in_specs=[pl.BlockSpec((D, D), lambda i: (0, 0)),
                  pl.BlockSpec((D, N_PAD // 8), lambda i: (0, i))],
        out_specs=pl.BlockSpec((D, N_PAD // 8), lambda i: (0, i)),
    )(W, xT)


def _dinv_body(p_ref, o_ref):
    deg = jnp.sum(p_ref[...], axis=0, keepdims=True)
    o_ref[...] = jnp.where(deg > 0, jax.lax.rsqrt(deg), 0.0)


def _dinv(partials):
    return pl.pallas_call(
        _dinv_body,
        out_shape=jax.ShapeDtypeStruct((1, N_PAD), jnp.float32),
        grid=(1,),
        in_specs=[pl.BlockSpec((TILES, N_PAD), lambda i: (0, 0))],
        out_specs=pl.BlockSpec((1, N_PAD), lambda i: (0, 0)),
    )(partials)


def _scale_body(v_ref, d_ref, o_ref):
    o_ref[...] = v_ref[...] * d_ref[...]


def _scale(vals, dinv_col):
    return pl.pallas_call(
        _scale_body,
        out_shape=jax.ShapeDtypeStruct((N, K), jnp.float32),
        grid=(5,),
        in_specs=[pl.BlockSpec((N // 5, K), lambda i: (i, 0)),
                  pl.BlockSpec((N // 5, 1), lambda i: (i, 0))],
        out_specs=pl.BlockSpec((N // 5, K), lambda i: (i, 0)),
    )(vals, dinv_col)


def _h2T_body(p_ref, di_ref, b_ref, w_ref, o_ref):
    hT = jnp.maximum(p_ref[...] * di_ref[...] + b_ref[...], 0.0)
    o_ref[...] = jax.lax.dot_general(
        w_ref[...], hT, (((0,), (0,)), ((), ())),
        preferred_element_type=jnp.float32)


def _h2T(o1T, dinv_row, b1_col, W2):
    return pl.pallas_call(
        _h2T_body,
        out_shape=jax.ShapeDtypeStruct((D, N_PAD), jnp.float32),
        grid=(8,),
        in_specs=[pl.BlockSpec((D, N_PAD // 8), lambda i: (0, i)),
                  pl.BlockSpec((1, N_PAD // 8), lambda i: (0, i)),
                  pl.BlockSpec((D, 1), lambda i: (0, 0)),
                  pl.BlockSpec((D, D), lambda i: (0, 0))],
        out_specs=pl.BlockSpec((D, N_PAD // 8), lambda i: (0, i)),
    )(o1T, dinv_row, b1_col, W2)


def _finT_body(p_ref, di_ref, b_ref, o_ref):
    o_ref[...] = jnp.transpose(
        p_ref[...] * di_ref[...] + b_ref[...], (1, 0))


def _finT(o2T, dinv_row, b2_col):
    return pl.pallas_call(
        _finT_body,
        out_shape=jax.ShapeDtypeStruct((N_PAD, D), jnp.float32),
        grid=(8,),
        in_specs=[pl.BlockSpec((D, N_PAD // 8), lambda i: (0, i)),
                  pl.BlockSpec((1, N_PAD // 8), lambda i: (0, i)),
                  pl.BlockSpec((D, 1), lambda i: (0, 0))],
        out_specs=pl.BlockSpec((N_PAD // 8, D), lambda i: (i, 0)),
    )(o2T, dinv_row, b2_col)


def _sc_mesh():
    return plsc.VectorSubcoreMesh(core_axis_name="c", subcore_axis_name="s")


def _sc_deg(idx_s, ew_s):
    @functools.partial(
        pl.kernel,
        mesh=_sc_mesh(),
        out_type=jax.ShapeDtypeStruct((TILES, N_PAD), jnp.float32),
        compiler_params=pltpu.CompilerParams(needs_layout_passes=False),
        scratch_types=[
            pltpu.VMEM((SRWS, SLOTS), jnp.int32),
            pltpu.VMEM((SRWS, SLOTS), jnp.float32),
            pltpu.VMEM((N_PAD,), jnp.float32),
        ],
    )
    def k(idx_hbm, ew_hbm, out_hbm, idx_v, ew_v, deg_v):
        w = lax.axis_index("s") * 2 + lax.axis_index("c")
        zero = jnp.zeros((16,), jnp.float32)

        def zloop(i, _):
            deg_v[pl.ds(i * 16, 16)] = zero
            return 0
        lax.fori_loop(0, N_PAD // 16, zloop, 0)

        for t in range(SGS // TILES):
            sg = w * (SGS // TILES) + t
            pltpu.sync_copy(idx_hbm.at[sg], idx_v)
            pltpu.sync_copy(ew_hbm.at[sg], ew_v)

            def row_loop(r, _):
                for half in range(SLOTS // 16):
                    dstv = idx_v[r, pl.ds(half * 16, 16)]
                    ewv = ew_v[r, pl.ds(half * 16, 16)]
                    plsc.addupdate_scatter(deg_v, [dstv], ewv,
                                           mask=dstv != DUMMY)
                return 0
            lax.fori_loop(0, SRWS, row_loop, 0)
        pltpu.sync_copy(deg_v, out_hbm.at[w])

    return k(idx_s, ew_s)


def _sc_conv(hwT, idx_s, ew_s):
    @functools.partial(
        pl.kernel,
        mesh=_sc_mesh(),
        out_type=jax.ShapeDtypeStruct((D, N_PAD), jnp.float32),
        compiler_params=pltpu.CompilerParams(needs_layout_passes=False),
        scratch_types=[
            pltpu.VMEM((FPT, N_PAD), jnp.float32),
            pltpu.VMEM((FPT, N_PAD), jnp.float32),
            pltpu.VMEM((SRWS, SLOTS), jnp.int32),
            pltpu.VMEM((SRWS, SLOTS), jnp.float32),
        ],
    )
    def k(hw_hbm, idx_hbm, ew_hbm, out_hbm, hw_v, acc_v, idx_v, ew_v):
        c = lax.axis_index("c")
        s = lax.axis_index("s")
        w = s * 2 + c
        pltpu.sync_copy(hw_hbm.at[pl.ds(w * FPT, FPT)], hw_v)
        zero = jnp.zeros((16,), jnp.float32)
        for f in range(FPT):
            def zloop(i, _, f=f):
                acc_v[f, pl.ds(i * 16, 16)] = zero
                return 0
            lax.fori_loop(0, N_PAD // 16, zloop, 0)

        def stage_loop(sg, _):
            pltpu.sync_copy(idx_hbm.at[sg], idx_v)
            pltpu.sync_copy(ew_hbm.at[sg], ew_v)

            def row_loop(r, _):
                grow = sg * SRWS + r
                gix = jnp.full((16,), grow, jnp.int32)
                hsp = [plsc.load_gather(
                    hw_v, [jnp.full((16,), f, jnp.int32), gix])
                    for f in range(FPT)]
                for half in range(SLOTS // 16):
                    dstv = idx_v[r, pl.ds(half * 16, 16)]
                    ewv = ew_v[r, pl.ds(half * 16, 16)]
                    m = dstv != DUMMY
                    for f in range(FPT):
                        plsc.addupdate_scatter(
                            acc_v,
                            [jnp.full((16,), f, jnp.int32), dstv],
                            ewv * hsp[f], mask=m)
                return 0
            lax.fori_loop(0, SRWS, row_loop, 0)
            return 0
        lax.fori_loop(0, SGS, stage_loop, 0)
        pltpu.sync_copy(acc_v, out_hbm.at[pl.ds(w * FPT, FPT)])

    return k(hwT, idx_s, ew_s)


def kernel(x, W1, b1, W2, b2):
    vals, idx = _knn_topk(x)
    idxp = jnp.pad(idx, ((0, N_PAD - N), (0, SLOTS - K)),
                   constant_values=DUMMY)
    xT_pad = jnp.pad(x, ((0, N_PAD - N), (0, 0))).T
    idx_s = idxp.reshape(SGS, SRWS, SLOTS)

    hw1T = _mmT(xT_pad, W1)
    ewp0 = jnp.pad(vals, ((0, N_PAD - N), (0, SLOTS - K)))
    degp = _sc_deg(idx_s, ewp0.reshape(SGS, SRWS, SLOTS))
    dinv = _dinv(degp)
    ew2 = _scale(vals, dinv.reshape(N_PAD)[:N].reshape(N, 1))
    ew_s = jnp.pad(ew2, ((0, N_PAD - N), (0, SLOTS - K))
                   ).reshape(SGS, SRWS, SLOTS)

    o1T = _sc_conv(hw1T, idx_s, ew_s)
    hw2T = _h2T(o1T, dinv, b1.reshape(D, 1), W2)
    o2T = _sc_conv(hw2T, idx_s, ew_s)
    return _finT(o2T, dinv, b2.reshape(D, 1))[:N]


# speedup vs baseline: 4.9285x; 1.0328x over previous
import functools

import jax
import jax.numpy as jnp
from jax import lax
from jax.experimental import pallas as pl
from jax.experimental.pallas import tpu as pltpu
from jax.experimental.pallas import tpu_sc as plsc

N = 10000
D = 128
K = 20
RB = 400
NEG = -3e38

N_PAD = 10240
TILES = 32
SLOTS = 32
SRWS = 160
SGS = N_PAD // SRWS
DUMMY = N
FPT = D // TILES


def _topk_body(xn_blk_ref, xn_all_ref, vals_ref, idx_ref, a_ref):
    xn_blk = xn_blk_ref[...]
    xn_all = xn_all_ref[...]
    a_ref[...] = jax.lax.dot_general(
        xn_blk, xn_all, (((1,), (1,)), ((), ())),
        preferred_element_type=jnp.float32)
    iota = jax.lax.broadcasted_iota(jnp.int32, (RB, N), 1)
    kiota = jax.lax.broadcasted_iota(jnp.int32, (RB, K), 1)
    vals_ref[...] = jnp.zeros((RB, K), jnp.float32)
    idx_ref[...] = jnp.zeros((RB, K), jnp.int32)

    def _round(k, _):
        a = a_ref[...]
        m = jnp.max(a, axis=1)
        am = jnp.min(jnp.where(a >= m[:, None], iota, N), axis=1)
        vals_ref[...] = jnp.where(kiota == k, m[:, None], vals_ref[...])
        idx_ref[...] = jnp.where(kiota == k, am[:, None], idx_ref[...])
        a_ref[...] = jnp.where(iota == am[:, None], NEG, a)
        return 0

    jax.lax.fori_loop(0, K, _round, 0)


def _norm_body(x_ref, xn_ref):
    x = x_ref[...]
    nrm = jnp.sqrt(jnp.sum(x * x, axis=1, keepdims=True))
    xn_ref[...] = x / jnp.maximum(nrm, 1e-12)


def _knn_topk(x):
    xn = pl.pallas_call(
        _norm_body,
        out_shape=jax.ShapeDtypeStruct((N, D), jnp.float32),
        grid=(5,),
        in_specs=[pl.BlockSpec((N // 5, D), lambda i: (i, 0))],
        out_specs=pl.BlockSpec((N // 5, D), lambda i: (i, 0)),
    )(x)
    vals, idx = pl.pallas_call(
        _topk_body,
        out_shape=(jax.ShapeDtypeStruct((N, K), jnp.float32),
                   jax.ShapeDtypeStruct((N, K), jnp.int32)),
        grid=(N // RB,),
        in_specs=[pl.BlockSpec((RB, D), lambda i: (i, 0)),
                  pl.BlockSpec((N, D), lambda i: (0, 0))],
        out_specs=(pl.BlockSpec((RB, K), lambda i: (i, 0)),
                   pl.BlockSpec((RB, K), lambda i: (i, 0))),
        scratch_shapes=[pltpu.VMEM((RB, N), jnp.float32)],
    )(xn, xn)
    return vals, idx


def _mmT_body(w_ref, xT_ref, o_ref):
    o_ref[...] = jax.lax.dot_general(
        w_ref[...], xT_ref[...], (((0,), (0,)), ((), ())),
        preferred_element_type=jnp.float32)


def _mmT(xT, W):
    return pl.pallas_call(
        _mmT_body,
        out_shape=jax.ShapeDtypeStruct((D, N_PAD), jnp.float32),
        grid=(8,),
        in_specs=[pl.BlockSpec((D, D), lambda i: (0, 0)),
                  pl.BlockSpec((D, N_PAD // 8), lambda i: (0, i))],
        out_specs=pl.BlockSpec((D, N_PAD // 8), lambda i: (0, i)),
    )(W, xT)


def _dinv_body(p_ref, o_ref):
    deg = jnp.sum(p_ref[...], axis=0, keepdims=True)
    o_ref[...] = jnp.where(deg > 0, jax.lax.rsqrt(deg), 0.0)


def _dinv(partials):
    return pl.pallas_call(
        _dinv_body,
        out_shape=jax.ShapeDtypeStruct((1, N_PAD), jnp.float32),
        grid=(1,),
        in_specs=[pl.BlockSpec((TILES, N_PAD), lambda i: (0, 0))],
        out_specs=pl.BlockSpec((1, N_PAD), lambda i: (0, 0)),
    )(partials)


def _scale_body(v_ref, d_ref, o_ref):
    o_ref[...] = v_ref[...] * d_ref[...]


def _scale(vals, dinv_col):
    return pl.pallas_call(
        _scale_body,
        out_shape=jax.ShapeDtypeStruct((N, K), jnp.float32),
        grid=(5,),
        in_specs=[pl.BlockSpec((N // 5, K), lambda i: (i, 0)),
                  pl.BlockSpec((N // 5, 1), lambda i: (i, 0))],
        out_specs=pl.BlockSpec((N // 5, K), lambda i: (i, 0)),
    )(vals, dinv_col)


def _h2T_body(p_ref, di_ref, b_ref, w_ref, o_ref):
    hT = jnp.maximum(p_ref[...] * di_ref[...] + b_ref[...], 0.0)
    o_ref[...] = jax.lax.dot_general(
        w_ref[...], hT, (((0,), (0,)), ((), ())),
        preferred_element_type=jnp.float32)


def _h2T(o1T, dinv_row, b1_col, W2):
    return pl.pallas_call(
        _h2T_body,
        out_shape=jax.ShapeDtypeStruct((D, N_PAD), jnp.float32),
        grid=(8,),
        in_specs=[pl.BlockSpec((D, N_PAD // 8), lambda i: (0, i)),
                  pl.BlockSpec((1, N_PAD // 8), lambda i: (0, i)),
                  pl.BlockSpec((D, 1), lambda i: (0, 0)),
                  pl.BlockSpec((D, D), lambda i: (0, 0))],
        out_specs=pl.BlockSpec((D, N_PAD // 8), lambda i: (0, i)),
    )(o1T, dinv_row, b1_col, W2)


def _finT_body(p_ref, di_ref, b_ref, o_ref):
    o_ref[...] = jnp.transpose(
        p_ref[...] * di_ref[...] + b_ref[...], (1, 0))


def _finT(o2T, dinv_row, b2_col):
    return pl.pallas_call(
        _finT_body,
        out_shape=jax.ShapeDtypeStruct((N_PAD, D), jnp.float32),
        grid=(8,),
        in_specs=[pl.BlockSpec((D, N_PAD // 8), lambda i: (0, i)),
                  pl.BlockSpec((1, N_PAD // 8), lambda i: (0, i)),
                  pl.BlockSpec((D, 1), lambda i: (0, 0))],
        out_specs=pl.BlockSpec((N_PAD // 8, D), lambda i: (i, 0)),
    )(o2T, dinv_row, b2_col)


def _sc_mesh():
    return plsc.VectorSubcoreMesh(core_axis_name="c", subcore_axis_name="s")


def _sc_deg(idx_s, ew_s):
    @functools.partial(
        pl.kernel,
        mesh=_sc_mesh(),
        out_type=jax.ShapeDtypeStruct((TILES, N_PAD), jnp.float32),
        compiler_params=pltpu.CompilerParams(needs_layout_passes=False),
        scratch_types=[
            pltpu.VMEM((SRWS, SLOTS), jnp.int32),
            pltpu.VMEM((SRWS, SLOTS), jnp.float32),
            pltpu.VMEM((N_PAD,), jnp.float32),
        ],
    )
    def k(idx_hbm, ew_hbm, out_hbm, idx_v, ew_v, deg_v):
        w = lax.axis_index("s") * 2 + lax.axis_index("c")
        zero = jnp.zeros((16,), jnp.float32)

        def zloop(i, _):
            deg_v[pl.ds(i * 16, 16)] = zero
            return 0
        lax.fori_loop(0, N_PAD // 16, zloop, 0)

        for t in range(SGS // TILES):
            sg = w * (SGS // TILES) + t
            pltpu.sync_copy(idx_hbm.at[sg], idx_v)
            pltpu.sync_copy(ew_hbm.at[sg], ew_v)

            def row_loop(r, _):
                for half in range(SLOTS // 16):
                    dstv = idx_v[r, pl.ds(half * 16, 16)]
                    ewv = ew_v[r, pl.ds(half * 16, 16)]
                    plsc.addupdate_scatter(deg_v, [dstv], ewv,
                                           mask=dstv != DUMMY)
                return 0
            lax.fori_loop(0, SRWS, row_loop, 0)
        pltpu.sync_copy(deg_v, out_hbm.at[w])

    return k(idx_s, ew_s)


def _sc_conv(hwT, idx_s, ew_s):
    @functools.partial(
        pl.kernel,
        mesh=_sc_mesh(),
        out_type=jax.ShapeDtypeStruct((D, N_PAD), jnp.float32),
        compiler_params=pltpu.CompilerParams(needs_layout_passes=False),
        scratch_types=[
            pltpu.VMEM((FPT, N_PAD), jnp.float32),
            pltpu.VMEM((FPT, N_PAD), jnp.float32),
            pltpu.VMEM((SRWS, SLOTS), jnp.int32),
            pltpu.VMEM((SRWS, SLOTS), jnp.float32),
        ],
    )
    def k(hw_hbm, idx_hbm, ew_hbm, out_hbm, hw_v, acc_v, idx_v, ew_v):
        c = lax.axis_index("c")
        s = lax.axis_index("s")
        w = s * 2 + c
        pltpu.sync_copy(hw_hbm.at[pl.ds(w * FPT, FPT)], hw_v)
        zero = jnp.zeros((16,), jnp.float32)
        for f in range(FPT):
            def zloop(i, _, f=f):
                acc_v[f, pl.ds(i * 16, 16)] = zero
                return 0
            lax.fori_loop(0, N_PAD // 16, zloop, 0)

        def stage_loop(sg, _):
            pltpu.sync_copy(idx_hbm.at[sg], idx_v)
            pltpu.sync_copy(ew_hbm.at[sg], ew_v)

            def row_loop(r, _):
                grow = sg * SRWS + r
                gix = jnp.full((16,), grow, jnp.int32)
                hsp = [plsc.load_gather(
                    hw_v, [jnp.full((16,), f, jnp.int32), gix])
                    for f in range(FPT)]
                for half in range(SLOTS // 16):
                    dstv = idx_v[r, pl.ds(half * 16, 16)]
                    ewv = ew_v[r, pl.ds(half * 16, 16)]
                    m = dstv != DUMMY
                    for f in range(FPT):
                        plsc.addupdate_scatter(
                            acc_v,
                            [jnp.full((16,), f, jnp.int32), dstv],
                            ewv * hsp[f], mask=m)
                return 0
            lax.fori_loop(0, SRWS, row_loop, 0)
            return 0
        lax.fori_loop(0, SGS, stage_loop, 0)
        pltpu.sync_copy(acc_v, out_hbm.at[pl.ds(w * FPT, FPT)])

    return k(hwT, idx_s, ew_s)


def kernel(x, W1, b1, W2, b2):
    vals, idx = _knn_topk(x)
    idxp = jnp.pad(idx, ((0, N_PAD - N), (0, SLOTS - K)),
                   constant_values=DUMMY)
    xT_pad = jnp.pad(x, ((0, N_PAD - N), (0, 0))).T
    idx_s = idxp.reshape(SGS, SRWS, SLOTS)

    hw1T = _mmT(xT_pad, W1)
    ewp0 = jnp.pad(vals, ((0, N_PAD - N), (0, SLOTS - K)))
    degp = _sc_deg(idx_s, ewp0.reshape(SGS, SRWS, SLOTS))
    dinv = _dinv(degp)
    ew2 = _scale(vals, dinv.reshape(N_PAD)[:N].reshape(N, 1))
    ew_s = jnp.pad(ew2, ((0, N_PAD - N), (0, SLOTS - K))
                   ).reshape(SGS, SRWS, SLOTS)

    o1T = _sc_conv(hw1T, idx_s, ew_s)
    hw2T = _h2T(o1T, dinv, b1.reshape(D, 1), W2)
    o2T = _sc_conv(hw2T, idx_s, ew_s)
    return _finT(o2T, dinv, b2.reshape(D, 1))[:N]


# speedup vs baseline: 5.0156x; 1.0177x over previous
import functools

import jax
import jax.numpy as jnp
from jax import lax
from jax.experimental import pallas as pl
from jax.experimental.pallas import tpu as pltpu
from jax.experimental.pallas import tpu_sc as plsc

N = 10000
D = 128
K = 20
RB = 400
NEG = -3e38

N_PAD = 10240
TILES = 32
SLOTS = 32
SRWS = 160
SGS = N_PAD // SRWS
DUMMY = N
FPT = D // TILES


def _topk_body(xn_blk_ref, xn_all_ref, vals_ref, idx_ref, a_ref):
    xn_blk = xn_blk_ref[...]
    xn_all = xn_all_ref[...]
    a_ref[...] = jax.lax.dot_general(
        xn_blk, xn_all, (((1,), (1,)), ((), ())),
        preferred_element_type=jnp.float32)
    iota = jax.lax.broadcasted_iota(jnp.int32, (RB, N), 1)
    kiota = jax.lax.broadcasted_iota(jnp.int32, (RB, K), 1)
    vals_ref[...] = jnp.zeros((RB, K), jnp.float32)
    idx_ref[...] = jnp.zeros((RB, K), jnp.int32)

    def _round(k, _):
        a = a_ref[...]
        m = jnp.max(a, axis=1)
        eq = a >= m[:, None]
        am = jnp.min(jnp.where(eq, iota, N), axis=1)
        vals_ref[...] = jnp.where(kiota == k, m[:, None], vals_ref[...])
        idx_ref[...] = jnp.where(kiota == k, am[:, None], idx_ref[...])
        a_ref[...] = jnp.where(eq, NEG, a)
        return 0

    jax.lax.fori_loop(0, K, _round, 0)


def _norm_body(x_ref, xn_ref):
    x = x_ref[...]
    nrm = jnp.sqrt(jnp.sum(x * x, axis=1, keepdims=True))
    xn_ref[...] = x / jnp.maximum(nrm, 1e-12)


def _knn_topk(x):
    xn = pl.pallas_call(
        _norm_body,
        out_shape=jax.ShapeDtypeStruct((N, D), jnp.float32),
        grid=(5,),
        in_specs=[pl.BlockSpec((N // 5, D), lambda i: (i, 0))],
        out_specs=pl.BlockSpec((N // 5, D), lambda i: (i, 0)),
    )(x)
    vals, idx = pl.pallas_call(
        _topk_body,
        out_shape=(jax.ShapeDtypeStruct((N, K), jnp.float32),
                   jax.ShapeDtypeStruct((N, K), jnp.int32)),
        grid=(N // RB,),
        in_specs=[pl.BlockSpec((RB, D), lambda i: (i, 0)),
                  pl.BlockSpec((N, D), lambda i: (0, 0))],
        out_specs=(pl.BlockSpec((RB, K), lambda i: (i, 0)),
                   pl.BlockSpec((RB, K), lambda i: (i, 0))),
        scratch_shapes=[pltpu.VMEM((RB, N), jnp.float32)],
    )(xn, xn)
    return vals, idx


def _mmT_body(w_ref, xT_ref, o_ref):
    o_ref[...] = jax.lax.dot_general(
        w_ref[...], xT_ref[...], (((0,), (0,)), ((), ())),
        preferred_element_type=jnp.float32)


def _mmT(xT, W):
    return pl.pallas_call(
        _mmT_body,
        out_shape=jax.ShapeDtypeStruct((D, N_PAD), jnp.float32),
        grid=(8,),
        in_specs=[pl.BlockSpec((D, D), lambda i: (0, 0)),
                  pl.BlockSpec((D, N_PAD // 8), lambda i: (0, i))],
        out_specs=pl.BlockSpec((D, N_PAD // 8), lambda i: (0, i)),
    )(W, xT)


def _dinv_body(p_ref, o_ref):
    deg = jnp.sum(p_ref[...], axis=0, keepdims=True)
    o_ref[...] = jnp.where(deg > 0, jax.lax.rsqrt(deg), 0.0)


def _dinv(partials):
    return pl.pallas_call(
        _dinv_body,
        out_shape=jax.ShapeDtypeStruct((1, N_PAD), jnp.float32),
        grid=(1,),
        in_specs=[pl.BlockSpec((TILES, N_PAD), lambda i: (0, 0))],
        out_specs=pl.BlockSpec((1, N_PAD), lambda i: (0, 0)),
    )(partials)


def _scale_body(v_ref, d_ref, o_ref):
    o_ref[...] = v_ref[...] * d_ref[...]


def _scale(vals, dinv_col):
    return pl.pallas_call(
        _scale_body,
        out_shape=jax.ShapeDtypeStruct((N, K), jnp.float32),
        grid=(5,),
        in_specs=[pl.BlockSpec((N // 5, K), lambda i: (i, 0)),
                  pl.BlockSpec((N // 5, 1), lambda i: (i, 0))],
        out_specs=pl.BlockSpec((N // 5, K), lambda i: (i, 0)),
    )(vals, dinv_col)


def _h2T_body(p_ref, di_ref, b_ref, w_ref, o_ref):
    hT = jnp.maximum(p_ref[...] * di_ref[...] + b_ref[...], 0.0)
    o_ref[...] = jax.lax.dot_general(
        w_ref[...], hT, (((0,), (0,)), ((), ())),
        preferred_element_type=jnp.float32)


def _h2T(o1T, dinv_row, b1_col, W2):
    return pl.pallas_call(
        _h2T_body,
        out_shape=jax.ShapeDtypeStruct((D, N_PAD), jnp.float32),
        grid=(8,),
        in_specs=[pl.BlockSpec((D, N_PAD // 8), lambda i: (0, i)),
                  pl.BlockSpec((1, N_PAD // 8), lambda i: (0, i)),
                  pl.BlockSpec((D, 1), lambda i: (0, 0)),
                  pl.BlockSpec((D, D), lambda i: (0, 0))],
        out_specs=pl.BlockSpec((D, N_PAD // 8), lambda i: (0, i)),
    )(o1T, dinv_row, b1_col, W2)


def _finT_body(p_ref, di_ref, b_ref, o_ref):
    o_ref[...] = jnp.transpose(
        p_ref[...] * di_ref[...] + b_ref[...], (1, 0))


def _finT(o2T, dinv_row, b2_col):
    return pl.pallas_call(
        _finT_body,
        out_shape=jax.ShapeDtypeStruct((N_PAD, D), jnp.float32),
        grid=(8,),
        in_specs=[pl.BlockSpec((D, N_PAD // 8), lambda i: (0, i)),
                  pl.BlockSpec((1, N_PAD // 8), lambda i: (0, i)),
                  pl.BlockSpec((D, 1), lambda i: (0, 0))],
        out_specs=pl.BlockSpec((N_PAD // 8, D), lambda i: (i, 0)),
    )(o2T, dinv_row, b2_col)


def _sc_mesh():
    return plsc.VectorSubcoreMesh(core_axis_name="c", subcore_axis_name="s")


def _sc_deg(idx_s, ew_s):
    @functools.partial(
        pl.kernel,
        mesh=_sc_mesh(),
        out_type=jax.ShapeDtypeStruct((TILES, N_PAD), jnp.float32),
        compiler_params=pltpu.CompilerParams(needs_layout_passes=False),
        scratch_types=[
            pltpu.VMEM((SRWS, SLOTS), jnp.int32),
            pltpu.VMEM((SRWS, SLOTS), jnp.float32),
            pltpu.VMEM((N_PAD,), jnp.float32),
        ],
    )
    def k(idx_hbm, ew_hbm, out_hbm, idx_v, ew_v, deg_v):
        w = lax.axis_index("s") * 2 + lax.axis_index("c")
        zero = jnp.zeros((16,), jnp.float32)

        def zloop(i, _):
            deg_v[pl.ds(i * 16, 16)] = zero
            return 0
        lax.fori_loop(0, N_PAD // 16, zloop, 0)

        for t in range(SGS // TILES):
            sg = w * (SGS // TILES) + t
            pltpu.sync_copy(idx_hbm.at[sg], idx_v)
            pltpu.sync_copy(ew_hbm.at[sg], ew_v)

            def row_loop(r, _):
                for half in range(SLOTS // 16):
                    dstv = idx_v[r, pl.ds(half * 16, 16)]
                    ewv = ew_v[r, pl.ds(half * 16, 16)]
                    plsc.addupdate_scatter(deg_v, [dstv], ewv,
                                           mask=dstv != DUMMY)
                return 0
            lax.fori_loop(0, SRWS, row_loop, 0)
        pltpu.sync_copy(deg_v, out_hbm.at[w])

    return k(idx_s, ew_s)


def _sc_conv(hwT, idx_s, ew_s):
    @functools.partial(
        pl.kernel,
        mesh=_sc_mesh(),
        out_type=jax.ShapeDtypeStruct((D, N_PAD), jnp.float32),
        compiler_params=pltpu.CompilerParams(needs_layout_passes=False),
        scratch_types=[
            pltpu.VMEM((FPT, N_PAD), jnp.float32),
            pltpu.VMEM((FPT, N_PAD), jnp.float32),
            pltpu.VMEM((SRWS, SLOTS), jnp.int32),
            pltpu.VMEM((SRWS, SLOTS), jnp.float32),
        ],
    )
    def k(hw_hbm, idx_hbm, ew_hbm, out_hbm, hw_v, acc_v, idx_v, ew_v):
        c = lax.axis_index("c")
        s = lax.axis_index("s")
        w = s * 2 + c
        pltpu.sync_copy(hw_hbm.at[pl.ds(w * FPT, FPT)], hw_v)
        zero = jnp.zeros((16,), jnp.float32)
        for f in range(FPT):
            def zloop(i, _, f=f):
                acc_v[f, pl.ds(i * 16, 16)] = zero
                return 0
            lax.fori_loop(0, N_PAD // 16, zloop, 0)

        def stage_loop(sg, _):
            pltpu.sync_copy(idx_hbm.at[sg], idx_v)
            pltpu.sync_copy(ew_hbm.at[sg], ew_v)

            def row_loop(r, _):
                grow = sg * SRWS + r
                gix = jnp.full((16,), grow, jnp.int32)
                hsp = [plsc.load_gather(
                    hw_v, [jnp.full((16,), f, jnp.int32), gix])
                    for f in range(FPT)]
                for half in range(SLOTS // 16):
                    dstv = idx_v[r, pl.ds(half * 16, 16)]
                    ewv = ew_v[r, pl.ds(half * 16, 16)]
                    m = dstv != DUMMY
                    for f in range(FPT):
                        plsc.addupdate_scatter(
                            acc_v,
                            [jnp.full((16,), f, jnp.int32), dstv],
                            ewv * hsp[f], mask=m)
                return 0
            lax.fori_loop(0, SRWS, row_loop, 0)
            return 0
        lax.fori_loop(0, SGS, stage_loop, 0)
        pltpu.sync_copy(acc_v, out_hbm.at[pl.ds(w * FPT, FPT)])

    return k(hwT, idx_s, ew_s)


def kernel(x, W1, b1, W2, b2):
    vals, idx = _knn_topk(x)
    idxp = jnp.pad(idx, ((0, N_PAD - N), (0, SLOTS - K)),
                   constant_values=DUMMY)
    xT_pad = jnp.pad(x, ((0, N_PAD - N), (0, 0))).T
    idx_s = idxp.reshape(SGS, SRWS, SLOTS)

    hw1T = _mmT(xT_pad, W1)
    ewp0 = jnp.pad(vals, ((0, N_PAD - N), (0, SLOTS - K)))
    degp = _sc_deg(idx_s, ewp0.reshape(SGS, SRWS, SLOTS))
    dinv = _dinv(degp)
    ew2 = _scale(vals, dinv.reshape(N_PAD)[:N].reshape(N, 1))
    ew_s = jnp.pad(ew2, ((0, N_PAD - N), (0, SLOTS - K))
                   ).reshape(SGS, SRWS, SLOTS)

    o1T = _sc_conv(hw1T, idx_s, ew_s)
    hw2T = _h2T(o1T, dinv, b1.reshape(D, 1), W2)
    o2T = _sc_conv(hw2T, idx_s, ew_s)
    return _finT(o2T, dinv, b2.reshape(D, 1))[:N]
